# baseline (speedup 1.0000x reference)
import jax
import jax.numpy as jnp
from jax.experimental import pallas as pl

_B, _V, _F = 8, 2048, 64
_K = 16
_NDIM = 4
_NPROP = 64
_NFILT = 128
_EXPF = 10.0
_RB = 512
_NRB = _V // _RB


def _gravnet_body(x_ref, xrow_ref, wflr_ref, bflr_ref, ws_ref, bs_ref,
                  wout_ref, bout_ref, out_ref):
    xb = x_ref[0]
    xr = xrow_ref[0]

    f_all = jnp.dot(xb, wflr_ref[...], preferred_element_type=jnp.float32) + bflr_ref[...]
    c_all = jnp.dot(xb, ws_ref[...], preferred_element_type=jnp.float32) + bs_ref[...]
    c_row = jnp.dot(xr, ws_ref[...], preferred_element_type=jnp.float32) + bs_ref[...]

    s = -2.0 * jax.lax.dot_general(c_row, c_all, (((1,), (1,)), ((), ())),
                                   preferred_element_type=jnp.float32)
    d_a = jnp.sum(c_row * c_row, axis=1, keepdims=True)
    d_b = jnp.sum(c_all * c_all, axis=1, keepdims=True)
    dist = jnp.abs(s + d_a + d_b.reshape(1, _V))

    iota = jax.lax.broadcasted_iota(jnp.int32, (_RB, _V), 1).astype(jnp.float32)
    inf = jnp.float32(jnp.inf)
    vf = jnp.float32(_V)

    f_bf = f_all.astype(jnp.bfloat16)

    work = dist
    sel_idx = []
    sel_d = []
    for _ in range(_K):
        m = jnp.min(work, axis=1, keepdims=True)
        cand = jnp.where(work == m, iota, vf)
        a = jnp.min(cand, axis=1, keepdims=True)
        work = jnp.where(cand == a, inf, work)
        sel_idx.append(a)
        sel_d.append(m)

    mean_acc = jnp.zeros((_RB, _NPROP), jnp.float32)
    max_acc = jnp.full((_RB, _NPROP), -inf, jnp.float32)
    for k in range(1, _K):
        one_hot = (iota == sel_idx[k]).astype(jnp.bfloat16)
        g = jnp.dot(one_hot, f_bf, preferred_element_type=jnp.float32)
        wg = g * jnp.exp(-_EXPF * sel_d[k])
        mean_acc = mean_acc + wg
        max_acc = jnp.maximum(max_acc, wg)
    agg_mean = mean_acc / jnp.float32(_K - 1)

    wout = wout_ref[...]
    out = (jnp.dot(xr, wout[:_F], preferred_element_type=jnp.float32)
           + jnp.dot(agg_mean, wout[_F:_F + _NPROP], preferred_element_type=jnp.float32)
           + jnp.dot(max_acc, wout[_F + _NPROP:], preferred_element_type=jnp.float32)
           + bout_ref[...])
    out_ref[0] = out


def kernel(x, W_flr, b_flr, W_s, b_s, W_out, b_out):
    grid = (_B, _NRB)
    out = pl.pallas_call(
        _gravnet_body,
        grid=grid,
        in_specs=[
            pl.BlockSpec((1, _V, _F), lambda b, r: (b, 0, 0)),
            pl.BlockSpec((1, _RB, _F), lambda b, r: (b, r, 0)),
            pl.BlockSpec((_F, _NPROP), lambda b, r: (0, 0)),
            pl.BlockSpec((1, _NPROP), lambda b, r: (0, 0)),
            pl.BlockSpec((_F, _NDIM), lambda b, r: (0, 0)),
            pl.BlockSpec((1, _NDIM), lambda b, r: (0, 0)),
            pl.BlockSpec((_F + 2 * _NPROP, _NFILT), lambda b, r: (0, 0)),
            pl.BlockSpec((1, _NFILT), lambda b, r: (0, 0)),
        ],
        out_specs=pl.BlockSpec((1, _RB, _NFILT), lambda b, r: (b, r, 0)),
        out_shape=jax.ShapeDtypeStruct((_B, _V, _NFILT), jnp.float32),
    )(x, x, W_flr, b_flr.reshape(1, _NPROP), W_s, b_s.reshape(1, _NDIM),
      W_out, b_out.reshape(1, _NFILT))
    return out


# speedup vs baseline: 1.0199x; 1.0199x over previous
import jax
import jax.numpy as jnp
from jax.experimental import pallas as pl

_B, _V, _F = 8, 2048, 64
_K = 16
_NDIM = 4
_NPROP = 64
_NFILT = 128
_EXPF = 10.0
_RB = 512
_NRB = _V // _RB


def _gravnet_body(x_ref, xrow_ref, wflr_ref, bflr_ref, ws_ref, bs_ref,
                  wout_ref, bout_ref, out_ref):
    xb = x_ref[0]
    xr = xrow_ref[0]

    f_all = jnp.dot(xb, wflr_ref[...], preferred_element_type=jnp.float32) + bflr_ref[...]
    c_all = jnp.dot(xb, ws_ref[...], preferred_element_type=jnp.float32) + bs_ref[...]
    c_row = jnp.dot(xr, ws_ref[...], preferred_element_type=jnp.float32) + bs_ref[...]

    s = -2.0 * jax.lax.dot_general(c_row, c_all, (((1,), (1,)), ((), ())),
                                   preferred_element_type=jnp.float32)
    d_a = jnp.sum(c_row * c_row, axis=1, keepdims=True)
    d_b = jnp.sum(c_all * c_all, axis=1, keepdims=True)
    dist = jnp.abs(s + d_a + d_b.reshape(1, _V))

    iota = jax.lax.broadcasted_iota(jnp.int32, (_RB, _V), 1).astype(jnp.float32)
    inf = jnp.float32(jnp.inf)
    vf = jnp.float32(_V)

    work = dist
    sel_idx = []
    sel_d = []
    for _ in range(_K):
        m = jnp.min(work, axis=1, keepdims=True)
        cand = jnp.where(work == m, iota, vf)
        a = jnp.min(cand, axis=1, keepdims=True)
        work = jnp.where(cand == a, inf, work)
        sel_idx.append(a)
        sel_d.append(m)

    mean_acc = jnp.zeros((_RB, _NPROP), jnp.float32)
    max_acc = jnp.full((_RB, _NPROP), -inf, jnp.float32)
    for k in range(1, _K):
        one_hot = (iota == sel_idx[k]).astype(jnp.float32)
        g = jnp.dot(one_hot, f_all, preferred_element_type=jnp.float32)
        wg = g * jnp.exp(-_EXPF * sel_d[k])
        mean_acc = mean_acc + wg
        max_acc = jnp.maximum(max_acc, wg)
    agg_mean = mean_acc / jnp.float32(_K - 1)

    wout = wout_ref[...]
    out = (jnp.dot(xr, wout[:_F], preferred_element_type=jnp.float32)
           + jnp.dot(agg_mean, wout[_F:_F + _NPROP], preferred_element_type=jnp.float32)
           + jnp.dot(max_acc, wout[_F + _NPROP:], preferred_element_type=jnp.float32)
           + bout_ref[...])
    out_ref[0] = out


def kernel(x, W_flr, b_flr, W_s, b_s, W_out, b_out):
    grid = (_B, _NRB)
    out = pl.pallas_call(
        _gravnet_body,
        grid=grid,
        in_specs=[
            pl.BlockSpec((1, _V, _F), lambda b, r: (b, 0, 0)),
            pl.BlockSpec((1, _RB, _F), lambda b, r: (b, r, 0)),
            pl.BlockSpec((_F, _NPROP), lambda b, r: (0, 0)),
            pl.BlockSpec((1, _NPROP), lambda b, r: (0, 0)),
            pl.BlockSpec((_F, _NDIM), lambda b, r: (0, 0)),
            pl.BlockSpec((1, _NDIM), lambda b, r: (0, 0)),
            pl.BlockSpec((_F + 2 * _NPROP, _NFILT), lambda b, r: (0, 0)),
            pl.BlockSpec((1, _NFILT), lambda b, r: (0, 0)),
        ],
        out_specs=pl.BlockSpec((1, _RB, _NFILT), lambda b, r: (b, r, 0)),
        out_shape=jax.ShapeDtypeStruct((_B, _V, _NFILT), jnp.float32),
    )(x, x, W_flr, b_flr.reshape(1, _NPROP), W_s, b_s.reshape(1, _NDIM),
      W_out, b_out.reshape(1, _NFILT))
    return out


# speedup vs baseline: 1.0944x; 1.0730x over previous
import jax
import jax.numpy as jnp
from jax.experimental import pallas as pl

_B, _V, _F = 8, 2048, 64
_K = 16
_NDIM = 4
_NPROP = 64
_NFILT = 128
_EXPF = 10.0
_RB = 512
_NRB = _V // _RB


def _gravnet_body(x_ref, xrow_ref, wflr_ref, bflr_ref, ws_ref, bs_ref,
                  wout_ref, bout_ref, out_ref):
    xb = x_ref[0]
    xr = xrow_ref[0]

    f_all = jnp.dot(xb, wflr_ref[...], preferred_element_type=jnp.float32) + bflr_ref[...]
    c_all = jnp.dot(xb, ws_ref[...], preferred_element_type=jnp.float32) + bs_ref[...]
    c_row = jnp.dot(xr, ws_ref[...], preferred_element_type=jnp.float32) + bs_ref[...]

    s = -2.0 * jax.lax.dot_general(c_row, c_all, (((1,), (1,)), ((), ())),
                                   preferred_element_type=jnp.float32)
    d_a = jnp.sum(c_row * c_row, axis=1, keepdims=True)
    d_b = jnp.sum(c_all * c_all, axis=1, keepdims=True)
    dist = jnp.abs(s + d_a + d_b.reshape(1, _V))

    iota = jax.lax.broadcasted_iota(jnp.int32, (_RB, _V), 1).astype(jnp.float32)
    inf = jnp.float32(jnp.inf)
    vf = jnp.float32(_V)

    work = dist
    sel_idx = []
    sel_d = []
    for _ in range(_K):
        m = jnp.min(work, axis=1, keepdims=True)
        cand = jnp.where(work == m, iota, vf)
        a = jnp.min(cand, axis=1, keepdims=True)
        work = jnp.where(iota == a, inf, work)
        sel_idx.append(a)
        sel_d.append(m)

    mean_acc = jnp.zeros((_RB, _NPROP), jnp.float32)
    max_acc = jnp.full((_RB, _NPROP), -inf, jnp.float32)
    for k in range(1, _K):
        one_hot = (iota == sel_idx[k]).astype(jnp.float32)
        g = jnp.dot(one_hot, f_all, preferred_element_type=jnp.float32)
        wg = g * jnp.exp(-_EXPF * sel_d[k])
        mean_acc = mean_acc + wg
        max_acc = jnp.maximum(max_acc, wg)
    agg_mean = mean_acc / jnp.float32(_K - 1)

    wout = wout_ref[...]
    out = (jnp.dot(xr, wout[:_F], preferred_element_type=jnp.float32)
           + jnp.dot(agg_mean, wout[_F:_F + _NPROP], preferred_element_type=jnp.float32)
           + jnp.dot(max_acc, wout[_F + _NPROP:], preferred_element_type=jnp.float32)
           + bout_ref[...])
    out_ref[0] = out


def kernel(x, W_flr, b_flr, W_s, b_s, W_out, b_out):
    grid = (_B, _NRB)
    out = pl.pallas_call(
        _gravnet_body,
        grid=grid,
        in_specs=[
            pl.BlockSpec((1, _V, _F), lambda b, r: (b, 0, 0)),
            pl.BlockSpec((1, _RB, _F), lambda b, r: (b, r, 0)),
            pl.BlockSpec((_F, _NPROP), lambda b, r: (0, 0)),
            pl.BlockSpec((1, _NPROP), lambda b, r: (0, 0)),
            pl.BlockSpec((_F, _NDIM), lambda b, r: (0, 0)),
            pl.BlockSpec((1, _NDIM), lambda b, r: (0, 0)),
            pl.BlockSpec((_F + 2 * _NPROP, _NFILT), lambda b, r: (0, 0)),
            pl.BlockSpec((1, _NFILT), lambda b, r: (0, 0)),
        ],
        out_specs=pl.BlockSpec((1, _RB, _NFILT), lambda b, r: (b, r, 0)),
        out_shape=jax.ShapeDtypeStruct((_B, _V, _NFILT), jnp.float32),
    )(x, x, W_flr, b_flr.reshape(1, _NPROP), W_s, b_s.reshape(1, _NDIM),
      W_out, b_out.reshape(1, _NFILT))
    return out
